# baseline (speedup 1.0000x reference)
import functools

import jax
import jax.numpy as jnp
from jax import lax
from jax.experimental import pallas as pl
from jax.experimental.pallas import tpu as pltpu
from jax.experimental.pallas import tpu_sc as plsc

N_NODES = 10000
N_EDGES = 320000
D = 128
OUT_D = 4 * D
NC = 2
NS = 16
NW = NC * NS
CH = 80

NBUF = 4
SNBUF = 4


def _sc_mesh():
    return plsc.VectorSubcoreMesh(
        core_axis_name="c", subcore_axis_name="s", num_cores=NC, num_subcores=NS)


@functools.cache
def _sc_gather(ne, ch, lo):
    epw = ne // NW
    n = epw // ch
    assert epw % ch == 0 and ch % 8 == 0 and ch <= 128 and lo % 8 == 0

    @functools.partial(
        pl.kernel,
        out_type=jax.ShapeDtypeStruct((ne, D), jnp.float32),
        scratch_types=[
            pltpu.VMEM((NBUF, ch), jnp.int32),
            pltpu.VMEM((NBUF, ch, D), jnp.float32),
            pltpu.SemaphoreType.DMA((NBUF,)),
            pltpu.SemaphoreType.DMA((NBUF,)),
            pltpu.SemaphoreType.DMA((NBUF,)),
        ],
        mesh=_sc_mesh(),
    )
    def gather_k(table_hbm, senders_hbm, out_hbm, idx_v, rows_v, semi, semg, semo):
        c = lax.axis_index("c")
        s = lax.axis_index("s")
        wid = s * NC + c

        def ebase(k):
            return pl.multiple_of(lo + wid * epw + k * ch, 8)

        def idx_copy(k, b):
            return pltpu.async_copy(
                senders_hbm.at[pl.ds(ebase(k), ch)], idx_v.at[b], semi.at[b])

        def gat_copy(k, b):
            return pltpu.async_copy(table_hbm.at[idx_v.at[b]], rows_v.at[b],
                                    semg.at[b])

        def out_copy(k, b):
            return pltpu.async_copy(rows_v.at[b], out_hbm.at[pl.ds(ebase(k), ch)],
                                    semo.at[b])

        idx_copy(0, 0)

        def body(i, _):
            b = lax.rem(i, NBUF)
            pltpu.make_async_copy(
                senders_hbm.at[pl.ds(ebase(i), ch)], idx_v.at[b], semi.at[b]).wait()
            gat_copy(i, b)

            @pl.when(i > 0)
            def _():
                bp = lax.rem(i + (NBUF - 1), NBUF)
                pltpu.make_async_copy(
                    table_hbm.at[idx_v.at[bp]], rows_v.at[bp], semg.at[bp]).wait()
                out_copy(i - 1, bp)

            @pl.when(i + 1 < n)
            def _():
                b1 = lax.rem(i + 1, NBUF)

                @pl.when(i + 1 >= NBUF)
                def _():
                    pltpu.make_async_copy(
                        rows_v.at[b1],
                        out_hbm.at[pl.ds(ebase(i + 1 - NBUF), ch)],
                        semo.at[b1]).wait()

                idx_copy(i + 1, b1)

            return 0

        lax.fori_loop(0, n, body, 0)
        bl = (n - 1) % NBUF
        pltpu.make_async_copy(
            table_hbm.at[idx_v.at[bl]], rows_v.at[bl], semg.at[bl]).wait()
        out_copy(n - 1, bl)
        for k in range(n - NBUF, n):
            if k >= 0:
                b = k % NBUF
                pltpu.make_async_copy(
                    rows_v.at[b], out_hbm.at[pl.ds(ebase(k), ch)], semo.at[b]).wait()

    return gather_k


BE = 2000


def _tc_body(vec_ref, rad_ref, gat_ref, w0_ref, w1_ref, w2_ref, w3_ref, out_ref):
    v = vec_ref[...]
    r = rad_ref[...]
    g = gat_ref[...]

    h = jnp.dot(r, w0_ref[...], preferred_element_type=jnp.float32)
    h = jax.nn.silu(h * (1.0 / jnp.sqrt(8.0)))
    h = jnp.dot(h, w1_ref[...], preferred_element_type=jnp.float32)
    h = jax.nn.silu(h * (1.0 / jnp.sqrt(64.0)))
    h = jnp.dot(h, w2_ref[...], preferred_element_type=jnp.float32)
    h = jax.nn.silu(h * (1.0 / jnp.sqrt(64.0)))
    mix = jnp.dot(h, w3_ref[...], preferred_element_type=jnp.float32)
    mix = mix * (1.0 / (jnp.sqrt(64.0) * jnp.sqrt(32.0)))

    rn = v * lax.rsqrt(jnp.sum(v * v, axis=1, keepdims=True) + 1e-12)
    sh = jnp.sqrt(3.0) * rn

    ms = g * mix[:, :D]
    mv = g * mix[:, D:]
    out_ref[:, 0:D] = ms
    out_ref[:, D:2 * D] = mv * sh[:, 0:1]
    out_ref[:, 2 * D:3 * D] = mv * sh[:, 1:2]
    out_ref[:, 3 * D:4 * D] = mv * sh[:, 2:3]


def _tc_messages(vectors, radial, gathered, W0, W1, W2, W3, ne, lo):
    grid = (ne // BE,)
    lob = lo // BE
    return pl.pallas_call(
        _tc_body,
        grid=grid,
        in_specs=[
            pl.BlockSpec((BE, 3), lambda i: (i + lob, 0)),
            pl.BlockSpec((BE, 8), lambda i: (i + lob, 0)),
            pl.BlockSpec((BE, D), lambda i: (i, 0)),
            pl.BlockSpec((8, 64), lambda i: (0, 0)),
            pl.BlockSpec((64, 64), lambda i: (0, 0)),
            pl.BlockSpec((64, 64), lambda i: (0, 0)),
            pl.BlockSpec((64, 256), lambda i: (0, 0)),
        ],
        out_specs=pl.BlockSpec((BE, OUT_D), lambda i: (i, 0)),
        out_shape=jax.ShapeDtypeStruct((ne, OUT_D), jnp.float32),
    )(vectors, radial, gathered, W0, W1, W2, W3)


_NPT = 624
_NPT_LAST = N_NODES - 15 * _NPT


@functools.cache
def _sc_scatter(ne, has_init, lo):
    ept = ne // NS
    n = ept // CH
    assert ept % CH == 0 and lo % 8 == 0

    def scatter_k(msg_hbm, recv_hbm, *rest):
        if has_init:
            init_hbm, out_hbm, idx_v, rows_v, zbuf, acc, semi, sema = rest
        else:
            init_hbm = None
            out_hbm, idx_v, rows_v, zbuf, acc, semi, sema = rest
        c = lax.axis_index("c")
        s = lax.axis_index("s")

        z16 = jnp.zeros((16,), jnp.float32)

        def zbody(i, _):
            for j in range(D // 16):
                zbuf[i, pl.ds(j * 16, 16)] = z16
            return 0

        if not has_init:
            lax.fori_loop(0, 16, zbody, 0)

        def slab(fn):
            @pl.when(s < 15)
            def _():
                fn(pl.multiple_of(s * _NPT, 8), _NPT)

            @pl.when(s == 15)
            def _():
                fn(15 * _NPT, _NPT_LAST)

        def zero_slab(base, m):
            def zb(i, _):
                pltpu.sync_copy(zbuf, acc.at[pl.ds(base + i * 16, 16)])
                return 0
            lax.fori_loop(0, m // 16, zb, 0)

        def ebase(k):
            return pl.multiple_of(lo + s * ept + k * CH, 8)

        def do_quarter(qcol):
            if has_init:
                slab(lambda base, m: pltpu.sync_copy(
                    init_hbm.at[pl.ds(base, m), pl.ds(qcol, D)],
                    acc.at[pl.ds(base, m)]))
            else:
                slab(zero_slab)
            plsc.subcore_barrier()

            def in_copies(k, b):
                pltpu.async_copy(recv_hbm.at[pl.ds(ebase(k), CH)], idx_v.at[b],
                                 semi.at[b])
                pltpu.async_copy(msg_hbm.at[pl.ds(ebase(k), CH), pl.ds(qcol, D)],
                                 rows_v.at[b], semi.at[b])

            def wait_in(k, b):
                pltpu.make_async_copy(recv_hbm.at[pl.ds(ebase(k), CH)],
                                      idx_v.at[b], semi.at[b]).wait()
                pltpu.make_async_copy(
                    msg_hbm.at[pl.ds(ebase(k), CH), pl.ds(qcol, D)],
                    rows_v.at[b], semi.at[b]).wait()

            def wait_add(b):
                pltpu.make_async_copy(rows_v.at[b], acc.at[idx_v.at[b]],
                                      sema.at[b]).wait()

            in_copies(0, 0)
            in_copies(1, 1)

            def body(i, _):
                b = lax.rem(i, SNBUF)
                wait_in(i, b)
                pltpu.async_copy(rows_v.at[b], acc.at[idx_v.at[b]], sema.at[b],
                                 add=True)

                @pl.when(i + 2 < n)
                def _():
                    b2 = lax.rem(i + 2, SNBUF)

                    @pl.when(i + 2 >= SNBUF)
                    def _():
                        wait_add(b2)

                    in_copies(i + 2, b2)

                return 0

            lax.fori_loop(0, n, body, 0)
            for k in range(n - SNBUF, n):
                wait_add(k % SNBUF)
            plsc.subcore_barrier()
            slab(lambda base, m: pltpu.sync_copy(
                acc.at[pl.ds(base, m)],
                out_hbm.at[pl.ds(base, m), pl.ds(qcol, D)]))
            plsc.subcore_barrier()

        for core in (0, 1):
            @pl.when(c == core)
            def _():
                do_quarter((2 * core) * D)
                do_quarter((2 * core + 1) * D)

    scratch = [
        pltpu.VMEM((SNBUF, CH), jnp.int32),
        pltpu.VMEM((SNBUF, CH, D), jnp.float32),
        pltpu.VMEM((16, D), jnp.float32),
        pltpu.VMEM_SHARED((N_NODES, D), jnp.float32),
        pltpu.SemaphoreType.DMA((SNBUF,)),
        pltpu.SemaphoreType.DMA((SNBUF,)),
    ]
    return pl.kernel(
        scatter_k,
        out_type=jax.ShapeDtypeStruct((N_NODES, OUT_D), jnp.float32),
        scratch_types=scratch,
        mesh=_sc_mesh(),
    )


def kernel(vectors, node_feats, radial_embedding, senders, receivers, W0, W1, W2, W3):
    out_cm = None
    for lo, hi, gch in ((0, 160000, 40), (160000, 320000, 40)):
        ne = hi - lo
        gathered = _sc_gather(ne, gch, lo)(node_feats, senders)
        msg_cm = _tc_messages(vectors, radial_embedding, gathered,
                              W0, W1, W2, W3, ne, lo)
        if out_cm is None:
            out_cm = _sc_scatter(ne, False, lo)(msg_cm, receivers)
        else:
            out_cm = _sc_scatter(ne, True, lo)(msg_cm, receivers, out_cm)
    out_s = out_cm[:, :D]
    out_v = out_cm[:, D:].reshape(N_NODES, 3, D).transpose(0, 2, 1).reshape(N_NODES, 3 * D)
    return jnp.concatenate([out_s, out_v], axis=1)


# speedup vs baseline: 1.0002x; 1.0002x over previous
import functools

import jax
import jax.numpy as jnp
from jax import lax
from jax.experimental import pallas as pl
from jax.experimental.pallas import tpu as pltpu
from jax.experimental.pallas import tpu_sc as plsc

N_NODES = 10000
N_EDGES = 320000
D = 128
OUT_D = 4 * D
NC = 2
NS = 16
NW = NC * NS
CH = 80

NBUF = 4
SNBUF = 4


def _sc_mesh():
    return plsc.VectorSubcoreMesh(
        core_axis_name="c", subcore_axis_name="s", num_cores=NC, num_subcores=NS)


@functools.cache
def _sc_gather(ne, ch, lo):
    epw = ne // NW
    n = epw // ch
    assert epw % ch == 0 and ch % 8 == 0 and ch <= 128 and lo % 8 == 0

    @functools.partial(
        pl.kernel,
        out_type=jax.ShapeDtypeStruct((ne, D), jnp.float32),
        scratch_types=[
            pltpu.VMEM((NBUF, ch), jnp.int32),
            pltpu.VMEM((NBUF, ch, D), jnp.float32),
            pltpu.SemaphoreType.DMA((NBUF,)),
            pltpu.SemaphoreType.DMA((NBUF,)),
            pltpu.SemaphoreType.DMA((NBUF,)),
        ],
        mesh=_sc_mesh(),
    )
    def gather_k(table_hbm, senders_hbm, out_hbm, idx_v, rows_v, semi, semg, semo):
        c = lax.axis_index("c")
        s = lax.axis_index("s")
        wid = s * NC + c

        def ebase(k):
            return pl.multiple_of(wid * epw + k * ch, 8)

        def gbase(k):
            return pl.multiple_of(lo + wid * epw + k * ch, 8)

        def idx_copy(k, b):
            return pltpu.async_copy(
                senders_hbm.at[pl.ds(gbase(k), ch)], idx_v.at[b], semi.at[b])

        def gat_copy(k, b):
            return pltpu.async_copy(table_hbm.at[idx_v.at[b]], rows_v.at[b],
                                    semg.at[b])

        def out_copy(k, b):
            return pltpu.async_copy(rows_v.at[b], out_hbm.at[pl.ds(ebase(k), ch)],
                                    semo.at[b])

        idx_copy(0, 0)

        def body(i, _):
            b = lax.rem(i, NBUF)
            pltpu.make_async_copy(
                senders_hbm.at[pl.ds(gbase(i), ch)], idx_v.at[b], semi.at[b]).wait()
            gat_copy(i, b)

            @pl.when(i > 0)
            def _():
                bp = lax.rem(i + (NBUF - 1), NBUF)
                pltpu.make_async_copy(
                    table_hbm.at[idx_v.at[bp]], rows_v.at[bp], semg.at[bp]).wait()
                out_copy(i - 1, bp)

            @pl.when(i + 1 < n)
            def _():
                b1 = lax.rem(i + 1, NBUF)

                @pl.when(i + 1 >= NBUF)
                def _():
                    pltpu.make_async_copy(
                        rows_v.at[b1],
                        out_hbm.at[pl.ds(ebase(i + 1 - NBUF), ch)],
                        semo.at[b1]).wait()

                idx_copy(i + 1, b1)

            return 0

        lax.fori_loop(0, n, body, 0)
        bl = (n - 1) % NBUF
        pltpu.make_async_copy(
            table_hbm.at[idx_v.at[bl]], rows_v.at[bl], semg.at[bl]).wait()
        out_copy(n - 1, bl)
        for k in range(n - NBUF, n):
            if k >= 0:
                b = k % NBUF
                pltpu.make_async_copy(
                    rows_v.at[b], out_hbm.at[pl.ds(ebase(k), ch)], semo.at[b]).wait()

    return gather_k


BE = 2000


def _tc_body(vec_ref, rad_ref, gat_ref, w0_ref, w1_ref, w2_ref, w3_ref, out_ref):
    v = vec_ref[...]
    r = rad_ref[...]
    g = gat_ref[...]

    h = jnp.dot(r, w0_ref[...], preferred_element_type=jnp.float32)
    h = jax.nn.silu(h * (1.0 / jnp.sqrt(8.0)))
    h = jnp.dot(h, w1_ref[...], preferred_element_type=jnp.float32)
    h = jax.nn.silu(h * (1.0 / jnp.sqrt(64.0)))
    h = jnp.dot(h, w2_ref[...], preferred_element_type=jnp.float32)
    h = jax.nn.silu(h * (1.0 / jnp.sqrt(64.0)))
    mix = jnp.dot(h, w3_ref[...], preferred_element_type=jnp.float32)
    mix = mix * (1.0 / (jnp.sqrt(64.0) * jnp.sqrt(32.0)))

    rn = v * lax.rsqrt(jnp.sum(v * v, axis=1, keepdims=True) + 1e-12)
    sh = jnp.sqrt(3.0) * rn

    ms = g * mix[:, :D]
    mv = g * mix[:, D:]
    out_ref[:, 0:D] = ms
    out_ref[:, D:2 * D] = mv * sh[:, 0:1]
    out_ref[:, 2 * D:3 * D] = mv * sh[:, 1:2]
    out_ref[:, 3 * D:4 * D] = mv * sh[:, 2:3]


def _tc_messages(vectors, radial, gathered, W0, W1, W2, W3, ne, lo):
    grid = (ne // BE,)
    lob = lo // BE
    return pl.pallas_call(
        _tc_body,
        grid=grid,
        in_specs=[
            pl.BlockSpec((BE, 3), lambda i: (i + lob, 0)),
            pl.BlockSpec((BE, 8), lambda i: (i + lob, 0)),
            pl.BlockSpec((BE, D), lambda i: (i, 0)),
            pl.BlockSpec((8, 64), lambda i: (0, 0)),
            pl.BlockSpec((64, 64), lambda i: (0, 0)),
            pl.BlockSpec((64, 64), lambda i: (0, 0)),
            pl.BlockSpec((64, 256), lambda i: (0, 0)),
        ],
        out_specs=pl.BlockSpec((BE, OUT_D), lambda i: (i, 0)),
        out_shape=jax.ShapeDtypeStruct((ne, OUT_D), jnp.float32),
    )(vectors, radial, gathered, W0, W1, W2, W3)


_NPT = 624
_NPT_LAST = N_NODES - 15 * _NPT


@functools.cache
def _sc_scatter(ne, has_init, lo):
    ept = ne // NS
    n = ept // CH
    assert ept % CH == 0 and lo % 8 == 0

    def scatter_k(msg_hbm, recv_hbm, *rest):
        if has_init:
            init_hbm, out_hbm, idx_v, rows_v, zbuf, acc, semi, sema = rest
        else:
            init_hbm = None
            out_hbm, idx_v, rows_v, zbuf, acc, semi, sema = rest
        c = lax.axis_index("c")
        s = lax.axis_index("s")

        z16 = jnp.zeros((16,), jnp.float32)

        def zbody(i, _):
            for j in range(D // 16):
                zbuf[i, pl.ds(j * 16, 16)] = z16
            return 0

        if not has_init:
            lax.fori_loop(0, 16, zbody, 0)

        def slab(fn):
            @pl.when(s < 15)
            def _():
                fn(pl.multiple_of(s * _NPT, 8), _NPT)

            @pl.when(s == 15)
            def _():
                fn(15 * _NPT, _NPT_LAST)

        def zero_slab(base, m):
            def zb(i, _):
                pltpu.sync_copy(zbuf, acc.at[pl.ds(base + i * 16, 16)])
                return 0
            lax.fori_loop(0, m // 16, zb, 0)

        def ebase(k):
            return pl.multiple_of(s * ept + k * CH, 8)

        def rbase(k):
            return pl.multiple_of(lo + s * ept + k * CH, 8)

        def do_quarter(qcol):
            if has_init:
                slab(lambda base, m: pltpu.sync_copy(
                    init_hbm.at[pl.ds(base, m), pl.ds(qcol, D)],
                    acc.at[pl.ds(base, m)]))
            else:
                slab(zero_slab)
            plsc.subcore_barrier()

            def in_copies(k, b):
                pltpu.async_copy(recv_hbm.at[pl.ds(rbase(k), CH)], idx_v.at[b],
                                 semi.at[b])
                pltpu.async_copy(msg_hbm.at[pl.ds(ebase(k), CH), pl.ds(qcol, D)],
                                 rows_v.at[b], semi.at[b])

            def wait_in(k, b):
                pltpu.make_async_copy(recv_hbm.at[pl.ds(rbase(k), CH)],
                                      idx_v.at[b], semi.at[b]).wait()
                pltpu.make_async_copy(
                    msg_hbm.at[pl.ds(ebase(k), CH), pl.ds(qcol, D)],
                    rows_v.at[b], semi.at[b]).wait()

            def wait_add(b):
                pltpu.make_async_copy(rows_v.at[b], acc.at[idx_v.at[b]],
                                      sema.at[b]).wait()

            in_copies(0, 0)
            in_copies(1, 1)

            def body(i, _):
                b = lax.rem(i, SNBUF)
                wait_in(i, b)
                pltpu.async_copy(rows_v.at[b], acc.at[idx_v.at[b]], sema.at[b],
                                 add=True)

                @pl.when(i + 2 < n)
                def _():
                    b2 = lax.rem(i + 2, SNBUF)

                    @pl.when(i + 2 >= SNBUF)
                    def _():
                        wait_add(b2)

                    in_copies(i + 2, b2)

                return 0

            lax.fori_loop(0, n, body, 0)
            for k in range(n - SNBUF, n):
                wait_add(k % SNBUF)
            plsc.subcore_barrier()
            slab(lambda base, m: pltpu.sync_copy(
                acc.at[pl.ds(base, m)],
                out_hbm.at[pl.ds(base, m), pl.ds(qcol, D)]))
            plsc.subcore_barrier()

        for core in (0, 1):
            @pl.when(c == core)
            def _():
                do_quarter((2 * core) * D)
                do_quarter((2 * core + 1) * D)

    scratch = [
        pltpu.VMEM((SNBUF, CH), jnp.int32),
        pltpu.VMEM((SNBUF, CH, D), jnp.float32),
        pltpu.VMEM((16, D), jnp.float32),
        pltpu.VMEM_SHARED((N_NODES, D), jnp.float32),
        pltpu.SemaphoreType.DMA((SNBUF,)),
        pltpu.SemaphoreType.DMA((SNBUF,)),
    ]
    return pl.kernel(
        scatter_k,
        out_type=jax.ShapeDtypeStruct((N_NODES, OUT_D), jnp.float32),
        scratch_types=scratch,
        mesh=_sc_mesh(),
    )


def kernel(vectors, node_feats, radial_embedding, senders, receivers, W0, W1, W2, W3):
    out_cm = None
    for lo, hi, gch in ((0, 160000, 40), (160000, 320000, 40)):
        ne = hi - lo
        gathered = _sc_gather(ne, gch, lo)(node_feats, senders)
        msg_cm = _tc_messages(vectors, radial_embedding, gathered,
                              W0, W1, W2, W3, ne, lo)
        if out_cm is None:
            out_cm = _sc_scatter(ne, False, lo)(msg_cm, receivers)
        else:
            out_cm = _sc_scatter(ne, True, lo)(msg_cm, receivers, out_cm)
    out_s = out_cm[:, :D]
    out_v = out_cm[:, D:].reshape(N_NODES, 3, D).transpose(0, 2, 1).reshape(N_NODES, 3 * D)
    return jnp.concatenate([out_s, out_v], axis=1)


# speedup vs baseline: 1.0210x; 1.0208x over previous
import functools

import jax
import jax.numpy as jnp
from jax import lax
from jax.experimental import pallas as pl
from jax.experimental.pallas import tpu as pltpu
from jax.experimental.pallas import tpu_sc as plsc

N_NODES = 10000
N_EDGES = 320000
D = 128
OUT_D = 4 * D
NC = 2
NS = 16
NW = NC * NS
CH = 80

NBUF = 4
SNBUF = 4


def _sc_mesh():
    return plsc.VectorSubcoreMesh(
        core_axis_name="c", subcore_axis_name="s", num_cores=NC, num_subcores=NS)


@functools.cache
def _sc_gather(ne, ch, lo):
    epw = ne // NW
    n = epw // ch
    assert epw % ch == 0 and ch % 8 == 0 and ch <= 128 and lo % 8 == 0

    @functools.partial(
        pl.kernel,
        out_type=jax.ShapeDtypeStruct((ne, D), jnp.float32),
        scratch_types=[
            pltpu.VMEM((NBUF, ch), jnp.int32),
            pltpu.VMEM((NBUF, ch, D), jnp.float32),
            pltpu.SemaphoreType.DMA((NBUF,)),
            pltpu.SemaphoreType.DMA((NBUF,)),
            pltpu.SemaphoreType.DMA((NBUF,)),
        ],
        mesh=_sc_mesh(),
    )
    def gather_k(table_hbm, senders_hbm, out_hbm, idx_v, rows_v, semi, semg, semo):
        c = lax.axis_index("c")
        s = lax.axis_index("s")
        wid = s * NC + c

        def ebase(k):
            return pl.multiple_of(wid * epw + k * ch, 8)

        def gbase(k):
            return pl.multiple_of(lo + wid * epw + k * ch, 8)

        def idx_copy(k, b):
            return pltpu.async_copy(
                senders_hbm.at[pl.ds(gbase(k), ch)], idx_v.at[b], semi.at[b])

        def gat_copy(k, b):
            return pltpu.async_copy(table_hbm.at[idx_v.at[b]], rows_v.at[b],
                                    semg.at[b])

        def out_copy(k, b):
            return pltpu.async_copy(rows_v.at[b], out_hbm.at[pl.ds(ebase(k), ch)],
                                    semo.at[b])

        idx_copy(0, 0)

        def body(i, _):
            b = lax.rem(i, NBUF)
            pltpu.make_async_copy(
                senders_hbm.at[pl.ds(gbase(i), ch)], idx_v.at[b], semi.at[b]).wait()
            gat_copy(i, b)

            @pl.when(i > 0)
            def _():
                bp = lax.rem(i + (NBUF - 1), NBUF)
                pltpu.make_async_copy(
                    table_hbm.at[idx_v.at[bp]], rows_v.at[bp], semg.at[bp]).wait()
                out_copy(i - 1, bp)

            @pl.when(i + 1 < n)
            def _():
                b1 = lax.rem(i + 1, NBUF)

                @pl.when(i + 1 >= NBUF)
                def _():
                    pltpu.make_async_copy(
                        rows_v.at[b1],
                        out_hbm.at[pl.ds(ebase(i + 1 - NBUF), ch)],
                        semo.at[b1]).wait()

                idx_copy(i + 1, b1)

            return 0

        lax.fori_loop(0, n, body, 0)
        bl = (n - 1) % NBUF
        pltpu.make_async_copy(
            table_hbm.at[idx_v.at[bl]], rows_v.at[bl], semg.at[bl]).wait()
        out_copy(n - 1, bl)
        for k in range(n - NBUF, n):
            if k >= 0:
                b = k % NBUF
                pltpu.make_async_copy(
                    rows_v.at[b], out_hbm.at[pl.ds(ebase(k), ch)], semo.at[b]).wait()

    return gather_k


BE = 2000


def _tc_body(vec_ref, rad_ref, gat_ref, w0_ref, w1_ref, w2_ref, w3_ref, out_ref):
    v = vec_ref[...]
    r = rad_ref[...]
    g = gat_ref[...]

    h = jnp.dot(r, w0_ref[...], preferred_element_type=jnp.float32)
    h = jax.nn.silu(h * (1.0 / jnp.sqrt(8.0)))
    h = jnp.dot(h, w1_ref[...], preferred_element_type=jnp.float32)
    h = jax.nn.silu(h * (1.0 / jnp.sqrt(64.0)))
    h = jnp.dot(h, w2_ref[...], preferred_element_type=jnp.float32)
    h = jax.nn.silu(h * (1.0 / jnp.sqrt(64.0)))
    mix = jnp.dot(h, w3_ref[...], preferred_element_type=jnp.float32)
    mix = mix * (1.0 / (jnp.sqrt(64.0) * jnp.sqrt(32.0)))

    rn = v * lax.rsqrt(jnp.sum(v * v, axis=1, keepdims=True) + 1e-12)
    sh = jnp.sqrt(3.0) * rn

    ms = g * mix[:, :D]
    mv = g * mix[:, D:]
    out_ref[:, 0:D] = ms
    out_ref[:, D:2 * D] = mv * sh[:, 0:1]
    out_ref[:, 2 * D:3 * D] = mv * sh[:, 1:2]
    out_ref[:, 3 * D:4 * D] = mv * sh[:, 2:3]


def _tc_messages(vectors, radial, gathered, W0, W1, W2, W3, ne, lo):
    grid = (ne // BE,)
    lob = lo // BE
    return pl.pallas_call(
        _tc_body,
        grid=grid,
        in_specs=[
            pl.BlockSpec((BE, 3), lambda i: (i + lob, 0)),
            pl.BlockSpec((BE, 8), lambda i: (i + lob, 0)),
            pl.BlockSpec((BE, D), lambda i: (i, 0)),
            pl.BlockSpec((8, 64), lambda i: (0, 0)),
            pl.BlockSpec((64, 64), lambda i: (0, 0)),
            pl.BlockSpec((64, 64), lambda i: (0, 0)),
            pl.BlockSpec((64, 256), lambda i: (0, 0)),
        ],
        out_specs=pl.BlockSpec((BE, OUT_D), lambda i: (i, 0)),
        out_shape=jax.ShapeDtypeStruct((ne, OUT_D), jnp.float32),
    )(vectors, radial, gathered, W0, W1, W2, W3)


_NPT = 624
_NPT_LAST = N_NODES - 15 * _NPT


@functools.cache
def _sc_scatter(ne, has_init, lo):
    ept = ne // NS
    n = ept // CH
    assert ept % CH == 0 and lo % 8 == 0

    def scatter_k(msg_hbm, recv_hbm, *rest):
        if has_init:
            init_hbm, out_hbm, idx_v, rows_v, zbuf, acc, semi, sema = rest
        else:
            init_hbm = None
            out_hbm, idx_v, rows_v, zbuf, acc, semi, sema = rest
        c = lax.axis_index("c")
        s = lax.axis_index("s")

        z16 = jnp.zeros((16,), jnp.float32)

        def zbody(i, _):
            for j in range(D // 16):
                zbuf[i, pl.ds(j * 16, 16)] = z16
            return 0

        if not has_init:
            lax.fori_loop(0, 16, zbody, 0)

        def slab(fn):
            @pl.when(s < 15)
            def _():
                fn(pl.multiple_of(s * _NPT, 8), _NPT)

            @pl.when(s == 15)
            def _():
                fn(15 * _NPT, _NPT_LAST)

        def zero_slab(base, m):
            def zb(i, _):
                pltpu.sync_copy(zbuf, acc.at[pl.ds(base + i * 16, 16)])
                return 0
            lax.fori_loop(0, m // 16, zb, 0)

        def ebase(k):
            return pl.multiple_of(s * ept + k * CH, 8)

        def rbase(k):
            return pl.multiple_of(lo + s * ept + k * CH, 8)

        def do_quarter(qcol):
            if has_init:
                slab(lambda base, m: pltpu.sync_copy(
                    init_hbm.at[pl.ds(base, m), pl.ds(qcol, D)],
                    acc.at[pl.ds(base, m)]))
            else:
                slab(zero_slab)
            plsc.subcore_barrier()

            def in_copies(k, b):
                pltpu.async_copy(recv_hbm.at[pl.ds(rbase(k), CH)], idx_v.at[b],
                                 semi.at[b])
                pltpu.async_copy(msg_hbm.at[pl.ds(ebase(k), CH), pl.ds(qcol, D)],
                                 rows_v.at[b], semi.at[b])

            def wait_in(k, b):
                pltpu.make_async_copy(recv_hbm.at[pl.ds(rbase(k), CH)],
                                      idx_v.at[b], semi.at[b]).wait()
                pltpu.make_async_copy(
                    msg_hbm.at[pl.ds(ebase(k), CH), pl.ds(qcol, D)],
                    rows_v.at[b], semi.at[b]).wait()

            def wait_add(b):
                pltpu.make_async_copy(rows_v.at[b], acc.at[idx_v.at[b]],
                                      sema.at[b]).wait()

            in_copies(0, 0)
            in_copies(1, 1)

            def body(i, _):
                b = lax.rem(i, SNBUF)
                wait_in(i, b)
                pltpu.async_copy(rows_v.at[b], acc.at[idx_v.at[b]], sema.at[b],
                                 add=True)

                @pl.when(i + 2 < n)
                def _():
                    b2 = lax.rem(i + 2, SNBUF)

                    @pl.when(i + 2 >= SNBUF)
                    def _():
                        wait_add(b2)

                    in_copies(i + 2, b2)

                return 0

            lax.fori_loop(0, n, body, 0)
            for k in range(n - SNBUF, n):
                wait_add(k % SNBUF)
            plsc.subcore_barrier()
            slab(lambda base, m: pltpu.sync_copy(
                acc.at[pl.ds(base, m)],
                out_hbm.at[pl.ds(base, m), pl.ds(qcol, D)]))
            plsc.subcore_barrier()

        for core in (0, 1):
            @pl.when(c == core)
            def _():
                do_quarter((2 * core) * D)
                do_quarter((2 * core + 1) * D)

    scratch = [
        pltpu.VMEM((SNBUF, CH), jnp.int32),
        pltpu.VMEM((SNBUF, CH, D), jnp.float32),
        pltpu.VMEM((16, D), jnp.float32),
        pltpu.VMEM_SHARED((N_NODES, D), jnp.float32),
        pltpu.SemaphoreType.DMA((SNBUF,)),
        pltpu.SemaphoreType.DMA((SNBUF,)),
    ]
    return pl.kernel(
        scatter_k,
        out_type=jax.ShapeDtypeStruct((N_NODES, OUT_D), jnp.float32),
        scratch_types=scratch,
        mesh=_sc_mesh(),
    )


def kernel(vectors, node_feats, radial_embedding, senders, receivers, W0, W1, W2, W3):
    out_cm = None
    for lo, hi, gch in ((0, 160000, 40), (160000, 320000, 40)):
        ne = hi - lo
        snd = lax.slice_in_dim(senders, lo, hi)
        rcv = lax.slice_in_dim(receivers, lo, hi)
        vec = lax.slice_in_dim(vectors, lo, hi)
        rad = lax.slice_in_dim(radial_embedding, lo, hi)
        gathered = _sc_gather(ne, gch, 0)(node_feats, snd)
        msg_cm = _tc_messages(vec, rad, gathered, W0, W1, W2, W3, ne, 0)
        if out_cm is None:
            out_cm = _sc_scatter(ne, False, 0)(msg_cm, rcv)
        else:
            out_cm = _sc_scatter(ne, True, 0)(msg_cm, rcv, out_cm)
    out_s = out_cm[:, :D]
    out_v = out_cm[:, D:].reshape(N_NODES, 3, D).transpose(0, 2, 1).reshape(N_NODES, 3 * D)
    return jnp.concatenate([out_s, out_v], axis=1)


# speedup vs baseline: 1.0294x; 1.0083x over previous
import functools

import jax
import jax.numpy as jnp
from jax import lax
from jax.experimental import pallas as pl
from jax.experimental.pallas import tpu as pltpu
from jax.experimental.pallas import tpu_sc as plsc

N_NODES = 10000
N_EDGES = 320000
D = 128
OUT_D = 4 * D
NC = 2
NS = 16
NW = NC * NS
CH = 80

NBUF = 4
SNBUF = 4


def _sc_mesh():
    return plsc.VectorSubcoreMesh(
        core_axis_name="c", subcore_axis_name="s", num_cores=NC, num_subcores=NS)


@functools.cache
def _sc_gather(ne, ch, lo):
    epw = ne // NW
    n = epw // ch
    assert epw % ch == 0 and ch % 8 == 0 and ch <= 128 and lo % 8 == 0

    @functools.partial(
        pl.kernel,
        out_type=jax.ShapeDtypeStruct((ne, D), jnp.float32),
        scratch_types=[
            pltpu.VMEM((NBUF, ch), jnp.int32),
            pltpu.VMEM((NBUF, ch, D), jnp.float32),
            pltpu.SemaphoreType.DMA((NBUF,)),
            pltpu.SemaphoreType.DMA((NBUF,)),
            pltpu.SemaphoreType.DMA((NBUF,)),
        ],
        mesh=_sc_mesh(),
    )
    def gather_k(table_hbm, senders_hbm, out_hbm, idx_v, rows_v, semi, semg, semo):
        c = lax.axis_index("c")
        s = lax.axis_index("s")
        wid = s * NC + c

        def ebase(k):
            return pl.multiple_of(wid * epw + k * ch, 8)

        def gbase(k):
            return pl.multiple_of(lo + wid * epw + k * ch, 8)

        def idx_copy(k, b):
            return pltpu.async_copy(
                senders_hbm.at[pl.ds(gbase(k), ch)], idx_v.at[b], semi.at[b])

        def gat_copy(k, b):
            return pltpu.async_copy(table_hbm.at[idx_v.at[b]], rows_v.at[b],
                                    semg.at[b])

        def out_copy(k, b):
            return pltpu.async_copy(rows_v.at[b], out_hbm.at[pl.ds(ebase(k), ch)],
                                    semo.at[b])

        idx_copy(0, 0)

        def body(i, _):
            b = lax.rem(i, NBUF)
            pltpu.make_async_copy(
                senders_hbm.at[pl.ds(gbase(i), ch)], idx_v.at[b], semi.at[b]).wait()
            gat_copy(i, b)

            @pl.when(i > 0)
            def _():
                bp = lax.rem(i + (NBUF - 1), NBUF)
                pltpu.make_async_copy(
                    table_hbm.at[idx_v.at[bp]], rows_v.at[bp], semg.at[bp]).wait()
                out_copy(i - 1, bp)

            @pl.when(i + 1 < n)
            def _():
                b1 = lax.rem(i + 1, NBUF)

                @pl.when(i + 1 >= NBUF)
                def _():
                    pltpu.make_async_copy(
                        rows_v.at[b1],
                        out_hbm.at[pl.ds(ebase(i + 1 - NBUF), ch)],
                        semo.at[b1]).wait()

                idx_copy(i + 1, b1)

            return 0

        lax.fori_loop(0, n, body, 0)
        bl = (n - 1) % NBUF
        pltpu.make_async_copy(
            table_hbm.at[idx_v.at[bl]], rows_v.at[bl], semg.at[bl]).wait()
        out_copy(n - 1, bl)
        for k in range(n - NBUF, n):
            if k >= 0:
                b = k % NBUF
                pltpu.make_async_copy(
                    rows_v.at[b], out_hbm.at[pl.ds(ebase(k), ch)], semo.at[b]).wait()

    return gather_k


BE = 4000


def _tc_body(vec_ref, rad_ref, gat_ref, w0_ref, w1_ref, w2_ref, w3_ref, out_ref):
    v = vec_ref[...]
    r = rad_ref[...]
    g = gat_ref[...]

    h = jnp.dot(r, w0_ref[...], preferred_element_type=jnp.float32)
    h = jax.nn.silu(h * (1.0 / jnp.sqrt(8.0)))
    h = jnp.dot(h, w1_ref[...], preferred_element_type=jnp.float32)
    h = jax.nn.silu(h * (1.0 / jnp.sqrt(64.0)))
    h = jnp.dot(h, w2_ref[...], preferred_element_type=jnp.float32)
    h = jax.nn.silu(h * (1.0 / jnp.sqrt(64.0)))
    mix = jnp.dot(h, w3_ref[...], preferred_element_type=jnp.float32)
    mix = mix * (1.0 / (jnp.sqrt(64.0) * jnp.sqrt(32.0)))

    rn = v * lax.rsqrt(jnp.sum(v * v, axis=1, keepdims=True) + 1e-12)
    sh = jnp.sqrt(3.0) * rn

    ms = g * mix[:, :D]
    mv = g * mix[:, D:]
    out_ref[:, 0:D] = ms
    out_ref[:, D:2 * D] = mv * sh[:, 0:1]
    out_ref[:, 2 * D:3 * D] = mv * sh[:, 1:2]
    out_ref[:, 3 * D:4 * D] = mv * sh[:, 2:3]


def _tc_messages(vectors, radial, gathered, W0, W1, W2, W3, ne, lo):
    grid = (ne // BE,)
    lob = lo // BE
    return pl.pallas_call(
        _tc_body,
        grid=grid,
        in_specs=[
            pl.BlockSpec((BE, 3), lambda i: (i + lob, 0)),
            pl.BlockSpec((BE, 8), lambda i: (i + lob, 0)),
            pl.BlockSpec((BE, D), lambda i: (i, 0)),
            pl.BlockSpec((8, 64), lambda i: (0, 0)),
            pl.BlockSpec((64, 64), lambda i: (0, 0)),
            pl.BlockSpec((64, 64), lambda i: (0, 0)),
            pl.BlockSpec((64, 256), lambda i: (0, 0)),
        ],
        out_specs=pl.BlockSpec((BE, OUT_D), lambda i: (i, 0)),
        out_shape=jax.ShapeDtypeStruct((ne, OUT_D), jnp.float32),
    )(vectors, radial, gathered, W0, W1, W2, W3)


_NPT = 624
_NPT_LAST = N_NODES - 15 * _NPT


@functools.cache
def _sc_scatter(ne, has_init, lo):
    ept = ne // NS
    n = ept // CH
    assert ept % CH == 0 and lo % 8 == 0

    def scatter_k(msg_hbm, recv_hbm, *rest):
        if has_init:
            init_hbm, out_hbm, idx_v, rows_v, zbuf, acc, semi, sema = rest
        else:
            init_hbm = None
            out_hbm, idx_v, rows_v, zbuf, acc, semi, sema = rest
        c = lax.axis_index("c")
        s = lax.axis_index("s")

        z16 = jnp.zeros((16,), jnp.float32)

        def zbody(i, _):
            for j in range(D // 16):
                zbuf[i, pl.ds(j * 16, 16)] = z16
            return 0

        if not has_init:
            lax.fori_loop(0, 16, zbody, 0)

        def slab(fn):
            @pl.when(s < 15)
            def _():
                fn(pl.multiple_of(s * _NPT, 8), _NPT)

            @pl.when(s == 15)
            def _():
                fn(15 * _NPT, _NPT_LAST)

        def zero_slab(base, m):
            def zb(i, _):
                pltpu.sync_copy(zbuf, acc.at[pl.ds(base + i * 16, 16)])
                return 0
            lax.fori_loop(0, m // 16, zb, 0)

        def ebase(k):
            return pl.multiple_of(s * ept + k * CH, 8)

        def rbase(k):
            return pl.multiple_of(lo + s * ept + k * CH, 8)

        def do_quarter(qcol):
            if has_init:
                slab(lambda base, m: pltpu.sync_copy(
                    init_hbm.at[pl.ds(base, m), pl.ds(qcol, D)],
                    acc.at[pl.ds(base, m)]))
            else:
                slab(zero_slab)
            plsc.subcore_barrier()

            def in_copies(k, b):
                pltpu.async_copy(recv_hbm.at[pl.ds(rbase(k), CH)], idx_v.at[b],
                                 semi.at[b])
                pltpu.async_copy(msg_hbm.at[pl.ds(ebase(k), CH), pl.ds(qcol, D)],
                                 rows_v.at[b], semi.at[b])

            def wait_in(k, b):
                pltpu.make_async_copy(recv_hbm.at[pl.ds(rbase(k), CH)],
                                      idx_v.at[b], semi.at[b]).wait()
                pltpu.make_async_copy(
                    msg_hbm.at[pl.ds(ebase(k), CH), pl.ds(qcol, D)],
                    rows_v.at[b], semi.at[b]).wait()

            def wait_add(b):
                pltpu.make_async_copy(rows_v.at[b], acc.at[idx_v.at[b]],
                                      sema.at[b]).wait()

            in_copies(0, 0)
            in_copies(1, 1)

            def body(i, _):
                b = lax.rem(i, SNBUF)
                wait_in(i, b)
                pltpu.async_copy(rows_v.at[b], acc.at[idx_v.at[b]], sema.at[b],
                                 add=True)

                @pl.when(i + 2 < n)
                def _():
                    b2 = lax.rem(i + 2, SNBUF)

                    @pl.when(i + 2 >= SNBUF)
                    def _():
                        wait_add(b2)

                    in_copies(i + 2, b2)

                return 0

            lax.fori_loop(0, n, body, 0)
            for k in range(n - SNBUF, n):
                wait_add(k % SNBUF)
            plsc.subcore_barrier()
            slab(lambda base, m: pltpu.sync_copy(
                acc.at[pl.ds(base, m)],
                out_hbm.at[pl.ds(base, m), pl.ds(qcol, D)]))
            plsc.subcore_barrier()

        for core in (0, 1):
            @pl.when(c == core)
            def _():
                do_quarter((2 * core) * D)
                do_quarter((2 * core + 1) * D)

    scratch = [
        pltpu.VMEM((SNBUF, CH), jnp.int32),
        pltpu.VMEM((SNBUF, CH, D), jnp.float32),
        pltpu.VMEM((16, D), jnp.float32),
        pltpu.VMEM_SHARED((N_NODES, D), jnp.float32),
        pltpu.SemaphoreType.DMA((SNBUF,)),
        pltpu.SemaphoreType.DMA((SNBUF,)),
    ]
    return pl.kernel(
        scatter_k,
        out_type=jax.ShapeDtypeStruct((N_NODES, OUT_D), jnp.float32),
        scratch_types=scratch,
        mesh=_sc_mesh(),
    )


def kernel(vectors, node_feats, radial_embedding, senders, receivers, W0, W1, W2, W3):
    out_cm = None
    for lo, hi, gch in ((0, 160000, 40), (160000, 320000, 40)):
        ne = hi - lo
        snd = lax.slice_in_dim(senders, lo, hi)
        rcv = lax.slice_in_dim(receivers, lo, hi)
        vec = lax.slice_in_dim(vectors, lo, hi)
        rad = lax.slice_in_dim(radial_embedding, lo, hi)
        gathered = _sc_gather(ne, gch, 0)(node_feats, snd)
        msg_cm = _tc_messages(vec, rad, gathered, W0, W1, W2, W3, ne, 0)
        if out_cm is None:
            out_cm = _sc_scatter(ne, False, 0)(msg_cm, rcv)
        else:
            out_cm = _sc_scatter(ne, True, 0)(msg_cm, rcv, out_cm)
    out_s = out_cm[:, :D]
    out_v = out_cm[:, D:].reshape(N_NODES, 3, D).transpose(0, 2, 1).reshape(N_NODES, 3 * D)
    return jnp.concatenate([out_s, out_v], axis=1)
